# baseline (speedup 1.0000x reference)
import functools

import jax
import jax.numpy as jnp
from jax import lax
from jax.experimental import pallas as pl
from jax.experimental.pallas import tpu as pltpu
from jax.experimental.pallas import tpu_sc as plsc

NC = 2
NS = 16
NW = NC * NS
LANES = 16
WIN = 128
K = 4

_mesh = plsc.VectorSubcoreMesh(
    core_axis_name="c", subcore_axis_name="s", num_cores=NC, num_subcores=NS
)
_params = pltpu.CompilerParams(use_tc_tiling_on_sc=False)


def _make_hist(NP, NWIN):
    rows = NP // NS
    G = 8
    assert NWIN % G == 0

    @functools.partial(
        pl.kernel,
        out_type=jax.ShapeDtypeStruct((NC, NP), jnp.float32),
        mesh=_mesh,
        compiler_params=_params,
        name="sc_hist",
        scratch_types=[
            pltpu.VMEM((NWIN, WIN), jnp.int32),
            pltpu.VMEM((WIN,), jnp.float32),
            pltpu.VMEM((rows,), jnp.float32),
            pltpu.VMEM_SHARED((NP,), jnp.float32),
            pltpu.SemaphoreType.DMA,
        ],
    )
    def hist(dstw, out, idx_v, ones_v, z_v, acc, sem):
        c = lax.axis_index("c")
        s = lax.axis_index("s")
        wid = c * NS + s

        def fill_ones(i, carry):
            ones_v[pl.ds(i * LANES, LANES)] = jnp.ones((LANES,), jnp.float32)
            return carry

        lax.fori_loop(0, WIN // LANES, fill_ones, 0)

        def fill_zero(i, carry):
            z_v[pl.ds(i * LANES, LANES)] = jnp.zeros((LANES,), jnp.float32)
            return carry

        lax.fori_loop(0, rows // LANES, fill_zero, 0)
        pltpu.sync_copy(z_v, acc.at[pl.ds(s * rows, rows)])
        pltpu.sync_copy(dstw.at[wid], idx_v)
        plsc.subcore_barrier()

        def body(gi, carry):
            for b in range(G):
                pltpu.async_copy(
                    ones_v, acc.at[idx_v.at[gi * G + b]], sem, add=True
                )
            for b in range(G):
                pltpu.make_async_copy(
                    ones_v, acc.at[idx_v.at[gi * G + b]], sem
                ).wait()
            return carry

        lax.fori_loop(0, NWIN // G, body, 0)
        plsc.subcore_barrier()
        pltpu.sync_copy(
            acc.at[pl.ds(s * rows, rows)], out.at[c, pl.ds(s * rows, rows)]
        )

    return hist


def _make_edge(NP, NWIN, F):
    rows = NP // NS
    NG2 = NWIN // (2 * K)
    assert NWIN == NG2 * 2 * K
    vshape = (WIN, F) if F else (WIN,)
    bufshape = (K,) + vshape
    accshape = (NP, F) if F else (NP,)
    zshape = (rows, F) if F else (rows,)
    outshape = (NC, NP, F) if F else (NC, NP)
    scratch = [
        pltpu.VMEM((NWIN, WIN), jnp.int32),
        pltpu.VMEM((NWIN, WIN), jnp.int32),
        pltpu.VMEM(bufshape, jnp.float32),
        pltpu.VMEM(bufshape, jnp.float32),
        pltpu.VMEM(zshape, jnp.float32),
        pltpu.VMEM_SHARED(accshape, jnp.float32),
        pltpu.SemaphoreType.DMA,
        pltpu.SemaphoreType.DMA,
        pltpu.SemaphoreType.DMA,
        pltpu.SemaphoreType.DMA,
    ]

    @functools.partial(
        pl.kernel,
        out_type=jax.ShapeDtypeStruct(outshape, jnp.float32),
        mesh=_mesh,
        compiler_params=_params,
        name="sc_edge_wide" if F else "sc_edge_scalar",
        scratch_types=scratch,
    )
    def edge(srcw, dstw, g, out, src_v, dst_v, buf0, buf1, z_v, acc,
             semg0, semg1, sems0, sems1):
        c = lax.axis_index("c")
        s = lax.axis_index("s")
        wid = c * NS + s
        gsrc = g

        if F:
            def fill_zero(i, carry):
                z_v[i] = jnp.zeros((F,), jnp.float32)
                return carry

            lax.fori_loop(0, rows, fill_zero, 0)
        else:
            def fill_zero(i, carry):
                z_v[pl.ds(i * LANES, LANES)] = jnp.zeros(
                    (LANES,), jnp.float32
                )
                return carry

            lax.fori_loop(0, rows // LANES, fill_zero, 0)
        pltpu.sync_copy(z_v, acc.at[pl.ds(s * rows, rows)])
        pltpu.sync_copy(srcw.at[wid], src_v)
        pltpu.sync_copy(dstw.at[wid], dst_v)
        plsc.subcore_barrier()

        def gath(j, buf, b, sem):
            pltpu.async_copy(gsrc.at[src_v.at[j]], buf.at[b], sem)

        def gath_wait(buf, b, sem):
            pltpu.make_async_copy(gsrc.at[src_v.at[0]], buf.at[b], sem).wait()

        def scat(j, buf, b, sem):
            pltpu.async_copy(buf.at[b], acc.at[dst_v.at[j]], sem, add=True)

        def scat_wait(buf, b, sem):
            pltpu.make_async_copy(
                buf.at[b], acc.at[dst_v.at[0]], sem
            ).wait()

        for b in range(K):
            gath(b, buf0, b, semg0)

        def pair(p, carry):
            j0 = 2 * p * K
            j1 = j0 + K

            @pl.when(p > 0)
            def _():
                for b in range(K):
                    scat_wait(buf1, b, sems1)

            for b in range(K):
                gath(j1 + b, buf1, b, semg1)
            for b in range(K):
                gath_wait(buf0, b, semg0)
            for b in range(K):
                scat(j0 + b, buf0, b, sems0)
            for b in range(K):
                scat_wait(buf0, b, sems0)

            @pl.when(p < NG2 - 1)
            def _():
                for b in range(K):
                    gath(j1 + K + b, buf0, b, semg0)

            for b in range(K):
                gath_wait(buf1, b, semg1)
            for b in range(K):
                scat(j1 + b, buf1, b, sems1)
            return carry

        lax.fori_loop(0, NG2, pair, 0)
        for b in range(K):
            scat_wait(buf1, b, sems1)
        plsc.subcore_barrier()
        pltpu.sync_copy(
            acc.at[pl.ds(s * rows, rows)], out.at[c, pl.ds(s * rows, rows)]
        )

    return edge


def _make_tca(NP, N, H):
    def body(xp, w1, hist_t, g1_out, dinv_out):
        deg = hist_t[:, 0:1] + hist_t[:, 1:2] + 1.0
        iota = lax.broadcasted_iota(jnp.int32, (NP, 1), 0)
        dinv = jnp.where(iota < N, lax.rsqrt(deg), 0.0)
        h1 = jnp.dot(xp[...], w1[...], preferred_element_type=jnp.float32)
        g1_out[...] = h1 * dinv
        dinv_out[...] = dinv

    return pl.pallas_call(
        body,
        out_shape=(
            jax.ShapeDtypeStruct((NP, H), jnp.float32),
            jax.ShapeDtypeStruct((NP, 1), jnp.float32),
        ),
    )


def _make_tcb(NP):
    def body(a0, a1, g1, dinv, b1r, w2, g2_out):
        s1 = a0[...] + a1[...] + g1[...]
        out1 = s1 * dinv[...] + b1r[...]
        r = jnp.maximum(out1, 0.0)
        h2 = jnp.dot(r, w2[...], preferred_element_type=jnp.float32)
        g2_out[...] = h2 * dinv[...]

    return pl.pallas_call(
        body, out_shape=jax.ShapeDtypeStruct((NP, 1), jnp.float32)
    )


def _make_tcc(NP):
    def body(agg2_t, g2, dinv, b2r, out):
        a = agg2_t[:, 0:1] + agg2_t[:, 1:2]
        z = (a + g2[...]) * dinv[...] + b2r[...]
        out[...] = jax.nn.sigmoid(z)

    return pl.pallas_call(
        body, out_shape=jax.ShapeDtypeStruct((NP, 1), jnp.float32)
    )


def kernel(x, edge_index, W1, b1, W2, b2):
    N, D = x.shape
    H = W1.shape[1]
    E = edge_index.shape[1]

    blk = NS * LANES * NC
    NP = (N // blk + 1) * blk
    EW = -(-E // NW)
    NWIN = -(-EW // WIN)
    NWIN = -(-NWIN // (2 * K)) * (2 * K)
    total = NW * NWIN * WIN

    src = edge_index[0].astype(jnp.int32)
    dst = edge_index[1].astype(jnp.int32)
    padn = total - E
    padidx = N + (jnp.arange(padn, dtype=jnp.int32) % (NP - N))
    srcp = jnp.concatenate([src, padidx]).reshape(NW, NWIN, WIN)
    dstp = jnp.concatenate([dst, padidx]).reshape(NW, NWIN, WIN)
    x_p = jnp.pad(x, ((0, NP - N), (0, 0)))
    b1r = b1.reshape(1, H)
    b2r = b2.reshape(1, 1)

    hist = _make_hist(NP, NWIN)(dstp)
    g1, dinv = _make_tca(NP, N, H)(x_p, W1, hist.T)
    agg1 = _make_edge(NP, NWIN, H)(srcp, dstp, g1)
    g2 = _make_tcb(NP)(agg1[0], agg1[1], g1, dinv, b1r, W2)
    agg2 = _make_edge(NP, NWIN, 0)(srcp, dstp, g2.reshape(NP))
    out = _make_tcc(NP)(agg2.T, g2, dinv, b2r)
    return out[:N]


# speedup vs baseline: 1.0516x; 1.0516x over previous
import functools

import jax
import jax.numpy as jnp
from jax import lax
from jax.experimental import pallas as pl
from jax.experimental.pallas import tpu as pltpu
from jax.experimental.pallas import tpu_sc as plsc

NC = 2
NS = 16
NW = NC * NS
LANES = 16
WIN = 128
K = 8

_mesh = plsc.VectorSubcoreMesh(
    core_axis_name="c", subcore_axis_name="s", num_cores=NC, num_subcores=NS
)
_params = pltpu.CompilerParams(use_tc_tiling_on_sc=False)


def _make_hist(NP, NWIN):
    rows = NP // NS
    G = 8
    assert NWIN % G == 0

    @functools.partial(
        pl.kernel,
        out_type=jax.ShapeDtypeStruct((NC, NP), jnp.float32),
        mesh=_mesh,
        compiler_params=_params,
        name="sc_hist",
        scratch_types=[
            pltpu.VMEM((NWIN, WIN), jnp.int32),
            pltpu.VMEM((WIN,), jnp.float32),
            pltpu.VMEM((rows,), jnp.float32),
            pltpu.VMEM_SHARED((NP,), jnp.float32),
            pltpu.SemaphoreType.DMA,
        ],
    )
    def hist(dstw, out, idx_v, ones_v, z_v, acc, sem):
        c = lax.axis_index("c")
        s = lax.axis_index("s")
        wid = c * NS + s

        def fill_ones(i, carry):
            ones_v[pl.ds(i * LANES, LANES)] = jnp.ones((LANES,), jnp.float32)
            return carry

        lax.fori_loop(0, WIN // LANES, fill_ones, 0)

        def fill_zero(i, carry):
            z_v[pl.ds(i * LANES, LANES)] = jnp.zeros((LANES,), jnp.float32)
            return carry

        lax.fori_loop(0, rows // LANES, fill_zero, 0)
        pltpu.sync_copy(z_v, acc.at[pl.ds(s * rows, rows)])
        pltpu.sync_copy(dstw.at[wid], idx_v)
        plsc.subcore_barrier()

        def body(gi, carry):
            for b in range(G):
                pltpu.async_copy(
                    ones_v, acc.at[idx_v.at[gi * G + b]], sem, add=True
                )
            for b in range(G):
                pltpu.make_async_copy(
                    ones_v, acc.at[idx_v.at[gi * G + b]], sem
                ).wait()
            return carry

        lax.fori_loop(0, NWIN // G, body, 0)
        plsc.subcore_barrier()
        pltpu.sync_copy(
            acc.at[pl.ds(s * rows, rows)], out.at[c, pl.ds(s * rows, rows)]
        )

    return hist


def _make_edge(NP, NWIN, F):
    rows = NP // NS
    NG2 = NWIN // (2 * K)
    assert NWIN == NG2 * 2 * K
    vshape = (WIN, F) if F else (WIN,)
    bufshape = (K,) + vshape
    accshape = (NP, F) if F else (NP,)
    zshape = (rows, F) if F else (rows,)
    outshape = (NC, NP, F) if F else (NC, NP)
    scratch = [
        pltpu.VMEM((NWIN, WIN), jnp.int32),
        pltpu.VMEM((NWIN, WIN), jnp.int32),
        pltpu.VMEM(bufshape, jnp.float32),
        pltpu.VMEM(bufshape, jnp.float32),
        pltpu.VMEM(zshape, jnp.float32),
        pltpu.VMEM_SHARED(accshape, jnp.float32),
        pltpu.SemaphoreType.DMA,
        pltpu.SemaphoreType.DMA,
        pltpu.SemaphoreType.DMA,
        pltpu.SemaphoreType.DMA,
    ]

    @functools.partial(
        pl.kernel,
        out_type=jax.ShapeDtypeStruct(outshape, jnp.float32),
        mesh=_mesh,
        compiler_params=_params,
        name="sc_edge_wide" if F else "sc_edge_scalar",
        scratch_types=scratch,
    )
    def edge(srcw, dstw, g, out, src_v, dst_v, buf0, buf1, z_v, acc,
             semg0, semg1, sems0, sems1):
        c = lax.axis_index("c")
        s = lax.axis_index("s")
        wid = c * NS + s
        gsrc = g

        if F:
            def fill_zero(i, carry):
                z_v[i] = jnp.zeros((F,), jnp.float32)
                return carry

            lax.fori_loop(0, rows, fill_zero, 0)
        else:
            def fill_zero(i, carry):
                z_v[pl.ds(i * LANES, LANES)] = jnp.zeros(
                    (LANES,), jnp.float32
                )
                return carry

            lax.fori_loop(0, rows // LANES, fill_zero, 0)
        pltpu.sync_copy(z_v, acc.at[pl.ds(s * rows, rows)])
        pltpu.sync_copy(srcw.at[wid], src_v)
        pltpu.sync_copy(dstw.at[wid], dst_v)
        plsc.subcore_barrier()

        def gath(j, buf, b, sem):
            pltpu.async_copy(gsrc.at[src_v.at[j]], buf.at[b], sem)

        def gath_wait(buf, b, sem):
            pltpu.make_async_copy(gsrc.at[src_v.at[0]], buf.at[b], sem).wait()

        def scat(j, buf, b, sem):
            pltpu.async_copy(buf.at[b], acc.at[dst_v.at[j]], sem, add=True)

        def scat_wait(buf, b, sem):
            pltpu.make_async_copy(
                buf.at[b], acc.at[dst_v.at[0]], sem
            ).wait()

        for b in range(K):
            gath(b, buf0, b, semg0)

        def pair(p, carry):
            j0 = 2 * p * K
            j1 = j0 + K

            @pl.when(p > 0)
            def _():
                for b in range(K):
                    scat_wait(buf1, b, sems1)

            for b in range(K):
                gath(j1 + b, buf1, b, semg1)
            for b in range(K):
                gath_wait(buf0, b, semg0)
            for b in range(K):
                scat(j0 + b, buf0, b, sems0)
            for b in range(K):
                scat_wait(buf0, b, sems0)

            @pl.when(p < NG2 - 1)
            def _():
                for b in range(K):
                    gath(j1 + K + b, buf0, b, semg0)

            for b in range(K):
                gath_wait(buf1, b, semg1)
            for b in range(K):
                scat(j1 + b, buf1, b, sems1)
            return carry

        lax.fori_loop(0, NG2, pair, 0)
        for b in range(K):
            scat_wait(buf1, b, sems1)
        plsc.subcore_barrier()
        pltpu.sync_copy(
            acc.at[pl.ds(s * rows, rows)], out.at[c, pl.ds(s * rows, rows)]
        )

    return edge


def _make_tca(NP, N, H):
    def body(xp, w1, hist_t, g1_out, dinv_out):
        deg = hist_t[:, 0:1] + hist_t[:, 1:2] + 1.0
        iota = lax.broadcasted_iota(jnp.int32, (NP, 1), 0)
        dinv = jnp.where(iota < N, lax.rsqrt(deg), 0.0)
        h1 = jnp.dot(xp[...], w1[...], preferred_element_type=jnp.float32,
                     precision=lax.Precision.HIGHEST)
        g1_out[...] = h1 * dinv
        dinv_out[...] = dinv

    return pl.pallas_call(
        body,
        out_shape=(
            jax.ShapeDtypeStruct((NP, H), jnp.float32),
            jax.ShapeDtypeStruct((NP, 1), jnp.float32),
        ),
    )


def _make_tcb(NP, H):
    def body(a0, a1, g1, dinv, b1r, q_out):
        s1 = a0[...] + a1[...] + g1[...]
        out1 = s1 * dinv[...] + b1r[...]
        r = jnp.maximum(out1, 0.0)
        q_out[...] = r * dinv[...]

    return pl.pallas_call(
        body, out_shape=jax.ShapeDtypeStruct((NP, H), jnp.float32)
    )


def _make_tcc(NP):
    def body(q0, q1, q, dinv, w2, b2r, out):
        t = q0[...] + q1[...] + q[...]
        h = jnp.dot(t, w2[...], preferred_element_type=jnp.float32,
                    precision=lax.Precision.HIGHEST)
        z = h * dinv[...] + b2r[...]
        out[...] = jax.nn.sigmoid(z)

    return pl.pallas_call(
        body, out_shape=jax.ShapeDtypeStruct((NP, 1), jnp.float32)
    )


def kernel(x, edge_index, W1, b1, W2, b2):
    N, D = x.shape
    H = W1.shape[1]
    E = edge_index.shape[1]

    blk = NS * LANES * NC
    NP = (N // blk + 1) * blk
    EW = -(-E // NW)
    NWIN = -(-EW // WIN)
    NWIN = -(-NWIN // (2 * K)) * (2 * K)
    total = NW * NWIN * WIN

    src = edge_index[0].astype(jnp.int32)
    dst = edge_index[1].astype(jnp.int32)
    padn = total - E
    padidx = N + (jnp.arange(padn, dtype=jnp.int32) % (NP - N))
    srcp = jnp.concatenate([src, padidx]).reshape(NW, NWIN, WIN)
    dstp = jnp.concatenate([dst, padidx]).reshape(NW, NWIN, WIN)
    x_p = jnp.pad(x, ((0, NP - N), (0, 0)))
    b1r = b1.reshape(1, H)
    b2r = b2.reshape(1, 1)

    edge_wide = _make_edge(NP, NWIN, H)
    hist = _make_hist(NP, NWIN)(dstp)
    g1, dinv = _make_tca(NP, N, H)(x_p, W1, hist.T)
    agg1 = edge_wide(srcp, dstp, g1)
    q = _make_tcb(NP, H)(agg1[0], agg1[1], g1, dinv, b1r)
    aggq = edge_wide(srcp, dstp, q)
    out = _make_tcc(NP)(aggq[0], aggq[1], q, dinv, W2, b2r)
    return out[:N]


# speedup vs baseline: 1.1722x; 1.1148x over previous
import functools

import jax
import jax.numpy as jnp
from jax import lax
from jax.experimental import pallas as pl
from jax.experimental.pallas import tpu as pltpu
from jax.experimental.pallas import tpu_sc as plsc

NC = 2
NS = 16
NW = NC * NS
LANES = 16
WIN = 128
K = 8

_mesh = plsc.VectorSubcoreMesh(
    core_axis_name="c", subcore_axis_name="s", num_cores=NC, num_subcores=NS
)
_params = pltpu.CompilerParams(use_tc_tiling_on_sc=False)


def _make_hist(NP, NWIN):
    rows = NP // NS
    G = 8
    assert NWIN % G == 0

    @functools.partial(
        pl.kernel,
        out_type=jax.ShapeDtypeStruct((NC, NP), jnp.float32),
        mesh=_mesh,
        compiler_params=_params,
        name="sc_hist",
        scratch_types=[
            pltpu.VMEM((NWIN, WIN), jnp.int32),
            pltpu.VMEM((WIN,), jnp.float32),
            pltpu.VMEM((rows,), jnp.float32),
            pltpu.VMEM_SHARED((NP,), jnp.float32),
            pltpu.SemaphoreType.DMA,
        ],
    )
    def hist(dstw, out, idx_v, ones_v, z_v, acc, sem):
        c = lax.axis_index("c")
        s = lax.axis_index("s")
        wid = c * NS + s

        def fill_ones(i, carry):
            ones_v[pl.ds(i * LANES, LANES)] = jnp.ones((LANES,), jnp.float32)
            return carry

        lax.fori_loop(0, WIN // LANES, fill_ones, 0)

        def fill_zero(i, carry):
            z_v[pl.ds(i * LANES, LANES)] = jnp.zeros((LANES,), jnp.float32)
            return carry

        lax.fori_loop(0, rows // LANES, fill_zero, 0)
        pltpu.sync_copy(z_v, acc.at[pl.ds(s * rows, rows)])
        pltpu.sync_copy(dstw.at[wid], idx_v)
        plsc.subcore_barrier()

        def body(gi, carry):
            for b in range(G):
                pltpu.async_copy(
                    ones_v, acc.at[idx_v.at[gi * G + b]], sem, add=True
                )
            for b in range(G):
                pltpu.make_async_copy(
                    ones_v, acc.at[idx_v.at[gi * G + b]], sem
                ).wait()
            return carry

        lax.fori_loop(0, NWIN // G, body, 0)
        plsc.subcore_barrier()
        pltpu.sync_copy(
            acc.at[pl.ds(s * rows, rows)], out.at[c, pl.ds(s * rows, rows)]
        )

    return hist


def _make_edge(NP, NWIN, F):
    rows = NP // NS
    NG2 = NWIN // (2 * K)
    assert NWIN == NG2 * 2 * K
    vshape = (WIN, F) if F else (WIN,)
    bufshape = (K,) + vshape
    accshape = (NP, F) if F else (NP,)
    zshape = (rows, F) if F else (rows,)
    outshape = (NC, NP, F) if F else (NC, NP)
    scratch = [
        pltpu.VMEM((NWIN, WIN), jnp.int32),
        pltpu.VMEM((NWIN, WIN), jnp.int32),
        pltpu.VMEM(bufshape, jnp.float32),
        pltpu.VMEM(bufshape, jnp.float32),
        pltpu.VMEM(zshape, jnp.float32),
        pltpu.VMEM_SHARED(accshape, jnp.float32),
        pltpu.SemaphoreType.DMA,
        pltpu.SemaphoreType.DMA,
        pltpu.SemaphoreType.DMA,
        pltpu.SemaphoreType.DMA,
    ]

    @functools.partial(
        pl.kernel,
        out_type=jax.ShapeDtypeStruct(outshape, jnp.float32),
        mesh=_mesh,
        compiler_params=_params,
        name="sc_edge_wide" if F else "sc_edge_scalar",
        scratch_types=scratch,
    )
    def edge(srcw, dstw, g, out, src_v, dst_v, buf0, buf1, z_v, acc,
             semg0, semg1, sems0, sems1):
        c = lax.axis_index("c")
        s = lax.axis_index("s")
        wid = c * NS + s
        gsrc = g

        if F:
            def fill_zero(i, carry):
                z_v[i] = jnp.zeros((F,), jnp.float32)
                return carry

            lax.fori_loop(0, rows, fill_zero, 0)
        else:
            def fill_zero(i, carry):
                z_v[pl.ds(i * LANES, LANES)] = jnp.zeros(
                    (LANES,), jnp.float32
                )
                return carry

            lax.fori_loop(0, rows // LANES, fill_zero, 0)
        pltpu.sync_copy(z_v, acc.at[pl.ds(s * rows, rows)])
        pltpu.sync_copy(srcw.at[wid], src_v)
        pltpu.sync_copy(dstw.at[wid], dst_v)
        plsc.subcore_barrier()

        def gath(j, buf, b, sem):
            pltpu.async_copy(gsrc.at[src_v.at[j]], buf.at[b], sem)

        def gath_wait(buf, b, sem):
            pltpu.make_async_copy(gsrc.at[src_v.at[0]], buf.at[b], sem).wait()

        def scat(j, buf, b, sem):
            pltpu.async_copy(buf.at[b], acc.at[dst_v.at[j]], sem, add=True)

        def scat_wait(buf, b, sem):
            pltpu.make_async_copy(
                buf.at[b], acc.at[dst_v.at[0]], sem
            ).wait()

        for b in range(K):
            gath(b, buf0, b, semg0)

        def pair(p, carry):
            j0 = 2 * p * K
            j1 = j0 + K

            @pl.when(p > 0)
            def _():
                for b in range(K):
                    scat_wait(buf1, b, sems1)

            for b in range(K):
                gath(j1 + b, buf1, b, semg1)
            for b in range(K):
                gath_wait(buf0, b, semg0)
            for b in range(K):
                scat(j0 + b, buf0, b, sems0)
            for b in range(K):
                scat_wait(buf0, b, sems0)

            @pl.when(p < NG2 - 1)
            def _():
                for b in range(K):
                    gath(j1 + K + b, buf0, b, semg0)

            for b in range(K):
                gath_wait(buf1, b, semg1)
            for b in range(K):
                scat(j1 + b, buf1, b, sems1)
            return carry

        lax.fori_loop(0, NG2, pair, 0)
        for b in range(K):
            scat_wait(buf1, b, sems1)
        plsc.subcore_barrier()
        pltpu.sync_copy(
            acc.at[pl.ds(s * rows, rows)], out.at[c, pl.ds(s * rows, rows)]
        )

    return edge


TB = 1280


def _make_tca(NP, N, H, D):
    nb = NP // TB

    def body(xp, w1, hist, ones21, g1_out, dinv_out):
        i = pl.program_id(0)
        deg = lax.dot_general(
            hist[...], ones21[...], (((0,), (0,)), ((), ())),
            preferred_element_type=jnp.float32,
        ) + 1.0
        iota = lax.broadcasted_iota(jnp.int32, (TB, 1), 0) + i * TB
        dinv = jnp.where(iota < N, lax.rsqrt(deg), 0.0)
        h1 = jnp.dot(xp[...], w1[...], preferred_element_type=jnp.float32,
                     precision=lax.Precision.HIGHEST)
        g1_out[...] = h1 * dinv
        dinv_out[...] = dinv

    return pl.pallas_call(
        body,
        grid=(nb,),
        in_specs=[
            pl.BlockSpec((TB, D), lambda i: (i, 0)),
            pl.BlockSpec((D, H), lambda i: (0, 0)),
            pl.BlockSpec((NC, TB), lambda i: (0, i)),
            pl.BlockSpec((NC, 1), lambda i: (0, 0)),
        ],
        out_specs=(
            pl.BlockSpec((TB, H), lambda i: (i, 0)),
            pl.BlockSpec((TB, 1), lambda i: (i, 0)),
        ),
        out_shape=(
            jax.ShapeDtypeStruct((NP, H), jnp.float32),
            jax.ShapeDtypeStruct((NP, 1), jnp.float32),
        ),
    )


def _make_tcb(NP, H):
    nb = NP // TB

    def body(agg, g1, dinv, b1r, q_out):
        s1 = agg[0] + agg[1] + g1[...]
        out1 = s1 * dinv[...] + b1r[...]
        r = jnp.maximum(out1, 0.0)
        q_out[...] = r * dinv[...]

    return pl.pallas_call(
        body,
        grid=(nb,),
        in_specs=[
            pl.BlockSpec((NC, TB, H), lambda i: (0, i, 0)),
            pl.BlockSpec((TB, H), lambda i: (i, 0)),
            pl.BlockSpec((TB, 1), lambda i: (i, 0)),
            pl.BlockSpec((1, H), lambda i: (0, 0)),
        ],
        out_specs=pl.BlockSpec((TB, H), lambda i: (i, 0)),
        out_shape=jax.ShapeDtypeStruct((NP, H), jnp.float32),
    )


def _make_tcc(NP, H):
    nb = NP // TB

    def body(agg, q, dinv, w2, b2r, out):
        t = agg[0] + agg[1] + q[...]
        h = jnp.dot(t, w2[...], preferred_element_type=jnp.float32,
                    precision=lax.Precision.HIGHEST)
        z = h * dinv[...] + b2r[...]
        out[...] = jax.nn.sigmoid(z)

    return pl.pallas_call(
        body,
        grid=(nb,),
        in_specs=[
            pl.BlockSpec((NC, TB, H), lambda i: (0, i, 0)),
            pl.BlockSpec((TB, H), lambda i: (i, 0)),
            pl.BlockSpec((TB, 1), lambda i: (i, 0)),
            pl.BlockSpec((H, 1), lambda i: (0, 0)),
            pl.BlockSpec((1, 1), lambda i: (0, 0)),
        ],
        out_specs=pl.BlockSpec((TB, 1), lambda i: (i, 0)),
        out_shape=jax.ShapeDtypeStruct((NP, 1), jnp.float32),
    )


def kernel(x, edge_index, W1, b1, W2, b2):
    N, D = x.shape
    H = W1.shape[1]
    E = edge_index.shape[1]

    blk = NS * LANES * NC
    NP = (N // blk + 1) * blk
    EW = -(-E // NW)
    NWIN = -(-EW // WIN)
    NWIN = -(-NWIN // (2 * K)) * (2 * K)
    total = NW * NWIN * WIN

    src = edge_index[0].astype(jnp.int32)
    dst = edge_index[1].astype(jnp.int32)
    padn = total - E
    padidx = N + (jnp.arange(padn, dtype=jnp.int32) % (NP - N))
    srcp = jnp.concatenate([src, padidx]).reshape(NW, NWIN, WIN)
    dstp = jnp.concatenate([dst, padidx]).reshape(NW, NWIN, WIN)
    x_p = jnp.pad(x, ((0, NP - N), (0, 0)))
    b1r = b1.reshape(1, H)
    b2r = b2.reshape(1, 1)
    ones21 = jnp.ones((NC, 1), jnp.float32)

    edge_wide = _make_edge(NP, NWIN, H)
    hist = _make_hist(NP, NWIN)(dstp)
    g1, dinv = _make_tca(NP, N, H, D)(x_p, W1, hist, ones21)
    agg1 = edge_wide(srcp, dstp, g1)
    q = _make_tcb(NP, H)(agg1, g1, dinv, b1r)
    aggq = edge_wide(srcp, dstp, q)
    out = _make_tcc(NP, H)(aggq, q, dinv, W2, b2r)
    return out[:N]


# speedup vs baseline: 1.5989x; 1.3640x over previous
import functools

import jax
import jax.numpy as jnp
from jax import lax
from jax.experimental import pallas as pl
from jax.experimental.pallas import tpu as pltpu
from jax.experimental.pallas import tpu_sc as plsc

NC = 2
NS = 16
NW = NC * NS
LANES = 16
WIN = 128
K = 8

_mesh = plsc.VectorSubcoreMesh(
    core_axis_name="c", subcore_axis_name="s", num_cores=NC, num_subcores=NS
)
_params = pltpu.CompilerParams(use_tc_tiling_on_sc=False)


def _make_hist(NP, NWIN):
    rows = NP // NS
    G = 8
    assert NWIN % G == 0

    @functools.partial(
        pl.kernel,
        out_type=jax.ShapeDtypeStruct((NC, NP), jnp.float32),
        mesh=_mesh,
        compiler_params=_params,
        name="sc_hist",
        scratch_types=[
            pltpu.VMEM((NWIN, WIN), jnp.int32),
            pltpu.VMEM((WIN,), jnp.float32),
            pltpu.VMEM((rows,), jnp.float32),
            pltpu.VMEM_SHARED((NP,), jnp.float32),
            pltpu.SemaphoreType.DMA,
        ],
    )
    def hist(dstw, out, idx_v, ones_v, z_v, acc, sem):
        c = lax.axis_index("c")
        s = lax.axis_index("s")
        wid = c * NS + s

        def fill_ones(i, carry):
            ones_v[pl.ds(i * LANES, LANES)] = jnp.ones((LANES,), jnp.float32)
            return carry

        lax.fori_loop(0, WIN // LANES, fill_ones, 0)

        def fill_zero(i, carry):
            z_v[pl.ds(i * LANES, LANES)] = jnp.zeros((LANES,), jnp.float32)
            return carry

        lax.fori_loop(0, rows // LANES, fill_zero, 0)
        pltpu.sync_copy(z_v, acc.at[pl.ds(s * rows, rows)])
        pltpu.sync_copy(dstw.at[wid], idx_v)
        plsc.subcore_barrier()

        def body(gi, carry):
            for b in range(G):
                pltpu.async_copy(
                    ones_v, acc.at[idx_v.at[gi * G + b]], sem, add=True
                )
            for b in range(G):
                pltpu.make_async_copy(
                    ones_v, acc.at[idx_v.at[gi * G + b]], sem
                ).wait()
            return carry

        lax.fori_loop(0, NWIN // G, body, 0)
        plsc.subcore_barrier()
        pltpu.sync_copy(
            acc.at[pl.ds(s * rows, rows)], out.at[c, pl.ds(s * rows, rows)]
        )

    return hist


def _make_edge(NP, NWIN, F):
    rows = NP // NS
    NG2 = NWIN // (2 * K)
    assert NWIN == NG2 * 2 * K
    vshape = (WIN, F) if F else (WIN,)
    bufshape = (K,) + vshape
    accshape = (NP, F) if F else (NP,)
    zshape = (rows, F) if F else (rows,)
    outshape = (NC, NP, F) if F else (NC, NP)
    scratch = [
        pltpu.VMEM((NWIN, WIN), jnp.int32),
        pltpu.VMEM((NWIN, WIN), jnp.int32),
        pltpu.VMEM(bufshape, jnp.float32),
        pltpu.VMEM(bufshape, jnp.float32),
        pltpu.VMEM(zshape, jnp.float32),
        pltpu.VMEM_SHARED(accshape, jnp.float32),
        pltpu.SemaphoreType.DMA,
        pltpu.SemaphoreType.DMA,
        pltpu.SemaphoreType.DMA,
        pltpu.SemaphoreType.DMA,
    ]

    @functools.partial(
        pl.kernel,
        out_type=jax.ShapeDtypeStruct(outshape, jnp.float32),
        mesh=_mesh,
        compiler_params=_params,
        name="sc_edge_wide" if F else "sc_edge_scalar",
        scratch_types=scratch,
    )
    def edge(srcw, dstw, g, out, src_v, dst_v, buf0, buf1, z_v, acc,
             semg0, semg1, sems0, sems1):
        c = lax.axis_index("c")
        s = lax.axis_index("s")
        wid = c * NS + s
        gsrc = g

        if F:
            def fill_zero(i, carry):
                z_v[i] = jnp.zeros((F,), jnp.float32)
                return carry

            lax.fori_loop(0, rows, fill_zero, 0)
        else:
            def fill_zero(i, carry):
                z_v[pl.ds(i * LANES, LANES)] = jnp.zeros(
                    (LANES,), jnp.float32
                )
                return carry

            lax.fori_loop(0, rows // LANES, fill_zero, 0)
        pltpu.sync_copy(z_v, acc.at[pl.ds(s * rows, rows)])
        pltpu.sync_copy(srcw.at[wid], src_v)
        pltpu.sync_copy(dstw.at[wid], dst_v)
        plsc.subcore_barrier()

        def gath(j, buf, b, sem):
            pltpu.async_copy(gsrc.at[src_v.at[j]], buf.at[b], sem)

        def gath_wait(buf, b, sem):
            pltpu.make_async_copy(gsrc.at[src_v.at[0]], buf.at[b], sem).wait()

        def scat(j, buf, b, sem):
            pltpu.async_copy(buf.at[b], acc.at[dst_v.at[j]], sem, add=True)

        def scat_wait(buf, b, sem):
            pltpu.make_async_copy(
                buf.at[b], acc.at[dst_v.at[0]], sem
            ).wait()

        for b in range(K):
            gath(b, buf0, b, semg0)

        def pair(p, carry):
            j0 = 2 * p * K
            j1 = j0 + K

            @pl.when(p > 0)
            def _():
                for b in range(K):
                    scat_wait(buf1, b, sems1)

            for b in range(K):
                gath(j1 + b, buf1, b, semg1)
            for b in range(K):
                gath_wait(buf0, b, semg0)
            for b in range(K):
                scat(j0 + b, buf0, b, sems0)
            for b in range(K):
                scat_wait(buf0, b, sems0)

            @pl.when(p < NG2 - 1)
            def _():
                for b in range(K):
                    gath(j1 + K + b, buf0, b, semg0)

            for b in range(K):
                gath_wait(buf1, b, semg1)
            for b in range(K):
                scat(j1 + b, buf1, b, sems1)
            return carry

        lax.fori_loop(0, NG2, pair, 0)
        for b in range(K):
            scat_wait(buf1, b, sems1)
        plsc.subcore_barrier()
        pltpu.sync_copy(
            acc.at[pl.ds(s * rows, rows)], out.at[c, pl.ds(s * rows, rows)]
        )

    return edge


RB = 160
PACK = 8


def _make_tca(NP, N, H, D):
    npk = NP // PACK
    nb = npk // RB
    DP = PACK * D

    def body(xp, w1p, hist8, s8, dinv8_out, dinvp_out, g1_out):
        i = pl.program_id(0)
        deg = hist8[0] + hist8[1] + 1.0
        row = lax.broadcasted_iota(jnp.int32, (RB, PACK), 0) + i * RB
        node = row * PACK + lax.broadcasted_iota(jnp.int32, (RB, PACK), 1)
        dinv8 = jnp.where(node < N, lax.rsqrt(deg), 0.0)
        dinvp = jnp.dot(dinv8, s8[...], preferred_element_type=jnp.float32)
        h1 = jnp.dot(xp[...], w1p[...], preferred_element_type=jnp.float32)
        dinv8_out[...] = dinv8
        dinvp_out[...] = dinvp
        g1_out[...] = h1 * dinvp

    return pl.pallas_call(
        body,
        grid=(nb,),
        in_specs=[
            pl.BlockSpec((RB, DP), lambda i: (i, 0)),
            pl.BlockSpec((DP, PACK * H), lambda i: (0, 0)),
            pl.BlockSpec((NC, RB, PACK), lambda i: (0, i, 0)),
            pl.BlockSpec((PACK, PACK * H), lambda i: (0, 0)),
        ],
        out_specs=(
            pl.BlockSpec((RB, PACK), lambda i: (i, 0)),
            pl.BlockSpec((RB, PACK * H), lambda i: (i, 0)),
            pl.BlockSpec((RB, PACK * H), lambda i: (i, 0)),
        ),
        out_shape=(
            jax.ShapeDtypeStruct((npk, PACK), jnp.float32),
            jax.ShapeDtypeStruct((npk, PACK * H), jnp.float32),
            jax.ShapeDtypeStruct((npk, PACK * H), jnp.float32),
        ),
    )


def _make_tcb(NP, H):
    npk = NP // PACK
    nb = npk // RB
    L = PACK * H

    def body(agg, g1, dinvp, b1p, q_out):
        s1 = agg[0] + agg[1] + g1[...]
        out1 = s1 * dinvp[...] + b1p[...]
        r = jnp.maximum(out1, 0.0)
        q_out[...] = r * dinvp[...]

    return pl.pallas_call(
        body,
        grid=(nb,),
        in_specs=[
            pl.BlockSpec((NC, RB, L), lambda i: (0, i, 0)),
            pl.BlockSpec((RB, L), lambda i: (i, 0)),
            pl.BlockSpec((RB, L), lambda i: (i, 0)),
            pl.BlockSpec((1, L), lambda i: (0, 0)),
        ],
        out_specs=pl.BlockSpec((RB, L), lambda i: (i, 0)),
        out_shape=jax.ShapeDtypeStruct((npk, L), jnp.float32),
    )


def _make_tcc(NP, H):
    npk = NP // PACK
    nb = npk // RB
    L = PACK * H

    def body(agg, q, dinv8, ws, b2r, out):
        t = agg[0] + agg[1] + q[...]
        h = jnp.dot(t, ws[...], preferred_element_type=jnp.float32)
        z = h * dinv8[...] + b2r[...]
        out[...] = jax.nn.sigmoid(z)

    return pl.pallas_call(
        body,
        grid=(nb,),
        in_specs=[
            pl.BlockSpec((NC, RB, L), lambda i: (0, i, 0)),
            pl.BlockSpec((RB, L), lambda i: (i, 0)),
            pl.BlockSpec((RB, PACK), lambda i: (i, 0)),
            pl.BlockSpec((L, PACK), lambda i: (0, 0)),
            pl.BlockSpec((1, 1), lambda i: (0, 0)),
        ],
        out_specs=pl.BlockSpec((RB, PACK), lambda i: (i, 0)),
        out_shape=jax.ShapeDtypeStruct((npk, PACK), jnp.float32),
    )


def kernel(x, edge_index, W1, b1, W2, b2):
    N, D = x.shape
    H = W1.shape[1]
    E = edge_index.shape[1]

    blk = NS * LANES * NC
    NP = (N // blk + 1) * blk
    EW = -(-E // NW)
    NWIN = -(-EW // WIN)
    NWIN = -(-NWIN // (2 * K)) * (2 * K)
    total = NW * NWIN * WIN

    src = edge_index[0].astype(jnp.int32)
    dst = edge_index[1].astype(jnp.int32)
    padn = total - E
    padidx = N + (jnp.arange(padn, dtype=jnp.int32) % (NP - N))
    srcp = jnp.concatenate([src, padidx]).reshape(NW, NWIN, WIN)
    dstp = jnp.concatenate([dst, padidx]).reshape(NW, NWIN, WIN)
    x_p = jnp.pad(x, ((0, NP - N), (0, 0)))
    npk = NP // PACK
    L = PACK * H

    eye8 = jnp.eye(PACK, dtype=jnp.float32)
    w1p = (eye8[:, None, :, None] * W1[None, :, None, :]).reshape(
        PACK * D, L
    )
    s8 = jnp.repeat(eye8, H, axis=1)
    ws = (eye8[:, None, :] * W2[None, :, 0:1]).reshape(L, PACK)
    b1p = jnp.tile(b1, PACK).reshape(1, L)
    b2r = b2.reshape(1, 1)
    x_pp = x_p.reshape(npk, PACK * D)

    edge_wide = _make_edge(NP, NWIN, H)
    hist = _make_hist(NP, NWIN)(dstp)
    hist8 = hist.reshape(NC, npk, PACK)
    dinv8, dinvp, g1p = _make_tca(NP, N, H, D)(x_pp, w1p, hist8, s8)
    agg1 = edge_wide(srcp, dstp, g1p.reshape(NP, H))
    q = _make_tcb(NP, H)(agg1.reshape(NC, npk, L), g1p, dinvp, b1p)
    aggq = edge_wide(srcp, dstp, q.reshape(NP, H))
    out8 = _make_tcc(NP, H)(aggq.reshape(NC, npk, L), q, dinv8, ws, b2r)
    return out8.reshape(NP, 1)[:N]


# speedup vs baseline: 1.6018x; 1.0018x over previous
import functools

import jax
import jax.numpy as jnp
from jax import lax
from jax.experimental import pallas as pl
from jax.experimental.pallas import tpu as pltpu
from jax.experimental.pallas import tpu_sc as plsc

NC = 2
NS = 16
NW = NC * NS
LANES = 16
WIN = 128
K = 10

_mesh = plsc.VectorSubcoreMesh(
    core_axis_name="c", subcore_axis_name="s", num_cores=NC, num_subcores=NS
)
_params = pltpu.CompilerParams(use_tc_tiling_on_sc=False)


def _make_hist(NP, NWIN):
    rows = NP // NS
    G = 8
    assert NWIN % G == 0

    @functools.partial(
        pl.kernel,
        out_type=jax.ShapeDtypeStruct((NC, NP), jnp.float32),
        mesh=_mesh,
        compiler_params=_params,
        name="sc_hist",
        scratch_types=[
            pltpu.VMEM((NWIN, WIN), jnp.int32),
            pltpu.VMEM((WIN,), jnp.float32),
            pltpu.VMEM((rows,), jnp.float32),
            pltpu.VMEM_SHARED((NP,), jnp.float32),
            pltpu.SemaphoreType.DMA,
        ],
    )
    def hist(dstw, out, idx_v, ones_v, z_v, acc, sem):
        c = lax.axis_index("c")
        s = lax.axis_index("s")
        wid = c * NS + s

        def fill_ones(i, carry):
            ones_v[pl.ds(i * LANES, LANES)] = jnp.ones((LANES,), jnp.float32)
            return carry

        lax.fori_loop(0, WIN // LANES, fill_ones, 0)

        def fill_zero(i, carry):
            z_v[pl.ds(i * LANES, LANES)] = jnp.zeros((LANES,), jnp.float32)
            return carry

        lax.fori_loop(0, rows // LANES, fill_zero, 0)
        pltpu.sync_copy(z_v, acc.at[pl.ds(s * rows, rows)])
        pltpu.sync_copy(dstw.at[wid], idx_v)
        plsc.subcore_barrier()

        def body(gi, carry):
            for b in range(G):
                pltpu.async_copy(
                    ones_v, acc.at[idx_v.at[gi * G + b]], sem, add=True
                )
            for b in range(G):
                pltpu.make_async_copy(
                    ones_v, acc.at[idx_v.at[gi * G + b]], sem
                ).wait()
            return carry

        lax.fori_loop(0, NWIN // G, body, 0)
        plsc.subcore_barrier()
        pltpu.sync_copy(
            acc.at[pl.ds(s * rows, rows)], out.at[c, pl.ds(s * rows, rows)]
        )

    return hist


def _make_edge(NP, NWIN, F):
    rows = NP // NS
    NG2 = NWIN // (2 * K)
    assert NWIN == NG2 * 2 * K
    vshape = (WIN, F) if F else (WIN,)
    bufshape = (K,) + vshape
    accshape = (NP, F) if F else (NP,)
    zshape = (rows, F) if F else (rows,)
    outshape = (NC, NP, F) if F else (NC, NP)
    scratch = [
        pltpu.VMEM((NWIN, WIN), jnp.int32),
        pltpu.VMEM((NWIN, WIN), jnp.int32),
        pltpu.VMEM(bufshape, jnp.float32),
        pltpu.VMEM(bufshape, jnp.float32),
        pltpu.VMEM(zshape, jnp.float32),
        pltpu.VMEM_SHARED(accshape, jnp.float32),
        pltpu.SemaphoreType.DMA,
        pltpu.SemaphoreType.DMA,
        pltpu.SemaphoreType.DMA,
        pltpu.SemaphoreType.DMA,
    ]

    @functools.partial(
        pl.kernel,
        out_type=jax.ShapeDtypeStruct(outshape, jnp.float32),
        mesh=_mesh,
        compiler_params=_params,
        name="sc_edge_wide" if F else "sc_edge_scalar",
        scratch_types=scratch,
    )
    def edge(srcw, dstw, g, out, src_v, dst_v, buf0, buf1, z_v, acc,
             semg0, semg1, sems0, sems1):
        c = lax.axis_index("c")
        s = lax.axis_index("s")
        wid = c * NS + s
        gsrc = g

        if F:
            def fill_zero(i, carry):
                z_v[i] = jnp.zeros((F,), jnp.float32)
                return carry

            lax.fori_loop(0, rows, fill_zero, 0)
        else:
            def fill_zero(i, carry):
                z_v[pl.ds(i * LANES, LANES)] = jnp.zeros(
                    (LANES,), jnp.float32
                )
                return carry

            lax.fori_loop(0, rows // LANES, fill_zero, 0)
        pltpu.sync_copy(z_v, acc.at[pl.ds(s * rows, rows)])
        pltpu.sync_copy(srcw.at[wid], src_v)
        pltpu.sync_copy(dstw.at[wid], dst_v)
        plsc.subcore_barrier()

        def gath(j, buf, b, sem):
            pltpu.async_copy(gsrc.at[src_v.at[j]], buf.at[b], sem)

        def gath_wait(buf, b, sem):
            pltpu.make_async_copy(gsrc.at[src_v.at[0]], buf.at[b], sem).wait()

        def scat(j, buf, b, sem):
            pltpu.async_copy(buf.at[b], acc.at[dst_v.at[j]], sem, add=True)

        def scat_wait(buf, b, sem):
            pltpu.make_async_copy(
                buf.at[b], acc.at[dst_v.at[0]], sem
            ).wait()

        for b in range(K):
            gath(b, buf0, b, semg0)

        def pair(p, carry):
            j0 = 2 * p * K
            j1 = j0 + K

            @pl.when(p > 0)
            def _():
                for b in range(K):
                    scat_wait(buf1, b, sems1)

            for b in range(K):
                gath(j1 + b, buf1, b, semg1)
            for b in range(K):
                gath_wait(buf0, b, semg0)
            for b in range(K):
                scat(j0 + b, buf0, b, sems0)
            for b in range(K):
                scat_wait(buf0, b, sems0)

            @pl.when(p < NG2 - 1)
            def _():
                for b in range(K):
                    gath(j1 + K + b, buf0, b, semg0)

            for b in range(K):
                gath_wait(buf1, b, semg1)
            for b in range(K):
                scat(j1 + b, buf1, b, sems1)
            return carry

        lax.fori_loop(0, NG2, pair, 0)
        for b in range(K):
            scat_wait(buf1, b, sems1)
        plsc.subcore_barrier()
        pltpu.sync_copy(
            acc.at[pl.ds(s * rows, rows)], out.at[c, pl.ds(s * rows, rows)]
        )

    return edge


RB = 160
PACK = 8


def _make_tca(NP, N, H, D):
    npk = NP // PACK
    nb = npk // RB
    DP = PACK * D

    def body(xp, w1p, hist8, s8, dinv8_out, dinvp_out, g1_out):
        i = pl.program_id(0)
        deg = hist8[0] + hist8[1] + 1.0
        row = lax.broadcasted_iota(jnp.int32, (RB, PACK), 0) + i * RB
        node = row * PACK + lax.broadcasted_iota(jnp.int32, (RB, PACK), 1)
        dinv8 = jnp.where(node < N, lax.rsqrt(deg), 0.0)
        dinvp = jnp.dot(dinv8, s8[...], preferred_element_type=jnp.float32)
        h1 = jnp.dot(xp[...], w1p[...], preferred_element_type=jnp.float32)
        dinv8_out[...] = dinv8
        dinvp_out[...] = dinvp
        g1_out[...] = h1 * dinvp

    return pl.pallas_call(
        body,
        grid=(nb,),
        in_specs=[
            pl.BlockSpec((RB, DP), lambda i: (i, 0)),
            pl.BlockSpec((DP, PACK * H), lambda i: (0, 0)),
            pl.BlockSpec((NC, RB, PACK), lambda i: (0, i, 0)),
            pl.BlockSpec((PACK, PACK * H), lambda i: (0, 0)),
        ],
        out_specs=(
            pl.BlockSpec((RB, PACK), lambda i: (i, 0)),
            pl.BlockSpec((RB, PACK * H), lambda i: (i, 0)),
            pl.BlockSpec((RB, PACK * H), lambda i: (i, 0)),
        ),
        out_shape=(
            jax.ShapeDtypeStruct((npk, PACK), jnp.float32),
            jax.ShapeDtypeStruct((npk, PACK * H), jnp.float32),
            jax.ShapeDtypeStruct((npk, PACK * H), jnp.float32),
        ),
    )


def _make_tcb(NP, H):
    npk = NP // PACK
    nb = npk // RB
    L = PACK * H

    def body(agg, g1, dinvp, b1p, q_out):
        s1 = agg[0] + agg[1] + g1[...]
        out1 = s1 * dinvp[...] + b1p[...]
        r = jnp.maximum(out1, 0.0)
        q_out[...] = r * dinvp[...]

    return pl.pallas_call(
        body,
        grid=(nb,),
        in_specs=[
            pl.BlockSpec((NC, RB, L), lambda i: (0, i, 0)),
            pl.BlockSpec((RB, L), lambda i: (i, 0)),
            pl.BlockSpec((RB, L), lambda i: (i, 0)),
            pl.BlockSpec((1, L), lambda i: (0, 0)),
        ],
        out_specs=pl.BlockSpec((RB, L), lambda i: (i, 0)),
        out_shape=jax.ShapeDtypeStruct((npk, L), jnp.float32),
    )


def _make_tcc(NP, H):
    npk = NP // PACK
    nb = npk // RB
    L = PACK * H

    def body(agg, q, dinv8, ws, b2r, out):
        t = agg[0] + agg[1] + q[...]
        h = jnp.dot(t, ws[...], preferred_element_type=jnp.float32)
        z = h * dinv8[...] + b2r[...]
        out[...] = jax.nn.sigmoid(z)

    return pl.pallas_call(
        body,
        grid=(nb,),
        in_specs=[
            pl.BlockSpec((NC, RB, L), lambda i: (0, i, 0)),
            pl.BlockSpec((RB, L), lambda i: (i, 0)),
            pl.BlockSpec((RB, PACK), lambda i: (i, 0)),
            pl.BlockSpec((L, PACK), lambda i: (0, 0)),
            pl.BlockSpec((1, 1), lambda i: (0, 0)),
        ],
        out_specs=pl.BlockSpec((RB, PACK), lambda i: (i, 0)),
        out_shape=jax.ShapeDtypeStruct((npk, PACK), jnp.float32),
    )


def kernel(x, edge_index, W1, b1, W2, b2):
    N, D = x.shape
    H = W1.shape[1]
    E = edge_index.shape[1]

    blk = NS * LANES * NC
    NP = (N // blk + 1) * blk
    EW = -(-E // NW)
    NWIN = -(-EW // WIN)
    NWIN = -(-NWIN // (2 * K)) * (2 * K)
    total = NW * NWIN * WIN

    src = edge_index[0].astype(jnp.int32)
    dst = edge_index[1].astype(jnp.int32)
    padn = total - E
    padidx = N + (jnp.arange(padn, dtype=jnp.int32) % (NP - N))
    srcp = jnp.concatenate([src, padidx]).reshape(NW, NWIN, WIN)
    dstp = jnp.concatenate([dst, padidx]).reshape(NW, NWIN, WIN)
    x_p = jnp.pad(x, ((0, NP - N), (0, 0)))
    npk = NP // PACK
    L = PACK * H

    eye8 = jnp.eye(PACK, dtype=jnp.float32)
    w1p = (eye8[:, None, :, None] * W1[None, :, None, :]).reshape(
        PACK * D, L
    )
    s8 = jnp.repeat(eye8, H, axis=1)
    ws = (eye8[:, None, :] * W2[None, :, 0:1]).reshape(L, PACK)
    b1p = jnp.tile(b1, PACK).reshape(1, L)
    b2r = b2.reshape(1, 1)
    x_pp = x_p.reshape(npk, PACK * D)

    edge_wide = _make_edge(NP, NWIN, H)
    hist = _make_hist(NP, NWIN)(dstp)
    hist8 = hist.reshape(NC, npk, PACK)
    dinv8, dinvp, g1p = _make_tca(NP, N, H, D)(x_pp, w1p, hist8, s8)
    agg1 = edge_wide(srcp, dstp, g1p.reshape(NP, H))
    q = _make_tcb(NP, H)(agg1.reshape(NC, npk, L), g1p, dinvp, b1p)
    aggq = edge_wide(srcp, dstp, q.reshape(NP, H))
    out8 = _make_tcc(NP, H)(aggq.reshape(NC, npk, L), q, dinv8, ws, b2r)
    return out8.reshape(NP, 1)[:N]
